# baseline (speedup 1.0000x reference)
import functools

import jax
import jax.numpy as jnp
from jax import lax
from jax.experimental import pallas as pl
from jax.experimental.pallas import tpu as pltpu
from jax.experimental.pallas import tpu_sc as plsc

N = 10000
IN_DIM = 128
E = 320000

NC, NS = 2, 16
K = 128
E_TOT = E + N
CHUNKS = -(-E_TOT // (NC * NS * K))
C0, C1 = 77, 85
EPT = CHUNKS * K
E_PAD = NC * NS * EPT
ROWS_PT = 632
N_ACC = NS * ROWS_PT
ZSIZES = (128, 128, 128, 128, 120)


def _sc_agg_body(h_hbm, as_hbm, ad_hbm, ids_hbm,
                 num_out, den_out,
                 num_sh, den_sh, idsb, dsc, hrows, msg, asv, adv, wv,
                 sem1, sem2, sem3, sem_sn, sem_sd):
    cid = lax.axis_index("c")
    sid = lax.axis_index("s")
    mychunks = jnp.where(cid == 0, C0, C1)
    blockbase = jnp.where(cid == 0, sid * C0, NS * C0 + sid * C1)

    def _zrow(r, carry):
        zero16 = jnp.zeros((16,), jnp.float32)
        for cb in range(8):
            msg[r, pl.ds(cb * 16, 16)] = zero16
        wv[r, :] = zero16
        return carry
    lax.fori_loop(0, K, _zrow, 0)
    zoff = 0
    for zs in ZSIZES:
        row0 = sid * ROWS_PT + zoff
        pltpu.sync_copy(msg.at[pl.ds(0, zs)], num_sh.at[pl.ds(row0, zs)])
        pltpu.sync_copy(wv.at[pl.ds(0, zs)], den_sh.at[pl.ds(row0, zs)])
        zoff += zs
    plsc.subcore_barrier()

    def _wait_num():
        pltpu.make_async_copy(msg, num_sh.at[dsc], sem_sn).wait()

    def _wait_den():
        pltpu.make_async_copy(wv, den_sh.at[dsc], sem_sd).wait()

    def _chunk(ci, carry):
        pltpu.sync_copy(ids_hbm.at[blockbase + ci], idsb)
        src_v = idsb.at[0]
        dst_v = idsb.at[1]
        cp1 = pltpu.async_copy(h_hbm.at[src_v], hrows, sem1)
        cp2 = pltpu.async_copy(as_hbm.at[src_v], asv, sem2)
        cp3 = pltpu.async_copy(ad_hbm.at[dst_v], adv, sem3)
        @pl.when(ci > 0)
        def _():
            _wait_num()
            _wait_den()
        cp2.wait()
        cp3.wait()
        for t in range(K // 16):
            dsc[pl.ds(t * 16, 16)] = idsb[1, pl.ds(t * 16, 16)]

        def _wrow(k, c):
            e = asv[k, :] + adv[k, :]
            wv[k, :] = jnp.exp(jnp.where(e > 0, e, e * 0.2))
            return c
        lax.fori_loop(0, K, _wrow, 0)
        pltpu.make_async_copy(wv, den_sh.at[dsc], sem_sd).start(add=True)
        cp1.wait()

        def _erow(k, c):
            wrow = wv[k, :]
            for hh in range(8):
                msg[k, pl.ds(hh * 16, 16)] = hrows[k, pl.ds(hh * 16, 16)] * wrow[hh]
            return c
        lax.fori_loop(0, K, _erow, 0)

        pltpu.make_async_copy(msg, num_sh.at[dsc], sem_sn).start(add=True)
        return carry
    lax.fori_loop(0, mychunks, _chunk, 0)
    _wait_num()
    _wait_den()

    plsc.subcore_barrier()
    zoff = 0
    for zs in ZSIZES:
        row0 = sid * ROWS_PT + zoff
        pltpu.sync_copy(num_sh.at[pl.ds(row0, zs)], msg.at[pl.ds(0, zs)])
        pltpu.sync_copy(msg.at[pl.ds(0, zs)], num_out.at[cid, pl.ds(row0, zs)])
        pltpu.sync_copy(den_sh.at[pl.ds(row0, zs)], wv.at[pl.ds(0, zs)])
        pltpu.sync_copy(wv.at[pl.ds(0, zs)], den_out.at[cid, pl.ds(row0, zs)])
        zoff += zs


_sc_agg = functools.partial(
    pl.kernel,
    out_type=[jax.ShapeDtypeStruct((NC, N_ACC, 128), jnp.float32),
              jax.ShapeDtypeStruct((NC, N_ACC, 16), jnp.float32)],
    mesh=plsc.VectorSubcoreMesh(core_axis_name="c", subcore_axis_name="s"),
    compiler_params=pltpu.CompilerParams(use_tc_tiling_on_sc=False),
    scratch_types=[
        pltpu.VMEM_SHARED((N_ACC, 128), jnp.float32),
        pltpu.VMEM_SHARED((N_ACC, 16), jnp.float32),
        pltpu.VMEM((2, K), jnp.int32),
        pltpu.VMEM((K,), jnp.int32),
        pltpu.VMEM((K, 128), jnp.float32),
        pltpu.VMEM((K, 128), jnp.float32),
        pltpu.VMEM((K, 16), jnp.float32),
        pltpu.VMEM((K, 16), jnp.float32),
        pltpu.VMEM((K, 16), jnp.float32),
        pltpu.SemaphoreType.DMA,
        pltpu.SemaphoreType.DMA,
        pltpu.SemaphoreType.DMA,
        pltpu.SemaphoreType.DMA,
        pltpu.SemaphoreType.DMA,
    ],
)(_sc_agg_body)


_GRID = 8
_BLK = N_ACC // _GRID


def _tc_enc_body(x_ref, w_ref, asm_ref, adm_ref, h_ref, as_ref, ad_ref):
    h = jnp.dot(x_ref[...], w_ref[...], preferred_element_type=jnp.float32)
    h_ref[...] = h
    as_ref[...] = jnp.dot(h, asm_ref[...], preferred_element_type=jnp.float32)
    ad_ref[...] = jnp.dot(h, adm_ref[...], preferred_element_type=jnp.float32)


def _merge(num_ref, den_ref, e16_ref, b_ref):
    nsum = num_ref[0] + num_ref[1]
    dsum = den_ref[0] + den_ref[1]
    recip = 1.0 / (dsum + 1e-16)
    rep = jnp.dot(recip, e16_ref[...], preferred_element_type=jnp.float32)
    return nsum * rep + b_ref[...]


def _tc_mid_body(apply_act, num_ref, den_ref, e16_ref, b_ref, w_ref, asm_ref,
                 adm_ref, h_ref, as_ref, ad_ref):
    act = _merge(num_ref, den_ref, e16_ref, b_ref)
    if apply_act:
        act = jnp.where(act > 0, act, act * 0.01)
    h = jnp.dot(act, w_ref[...], preferred_element_type=jnp.float32)
    h_ref[...] = h
    as_ref[...] = jnp.dot(h, asm_ref[...], preferred_element_type=jnp.float32)
    ad_ref[...] = jnp.dot(h, adm_ref[...], preferred_element_type=jnp.float32)


def _tc_final_body(num_ref, den_ref, e16_ref, b_ref, out_ref):
    out_ref[...] = _merge(num_ref, den_ref, e16_ref, b_ref)


def _rowspec(minor):
    return pl.BlockSpec((_BLK, minor), lambda i: (i, 0))


def _accspec(minor):
    return pl.BlockSpec((NC, _BLK, minor), lambda i: (0, i, 0))


def _fullspec(shape):
    return pl.BlockSpec(shape, lambda i: tuple(0 for _ in shape))


_tabs_shape = [jax.ShapeDtypeStruct((N_ACC, 128), jnp.float32),
               jax.ShapeDtypeStruct((N_ACC, 16), jnp.float32),
               jax.ShapeDtypeStruct((N_ACC, 16), jnp.float32)]
_tabs_spec = [_rowspec(128), _rowspec(16), _rowspec(16)]

_tc_enc = pl.pallas_call(
    _tc_enc_body,
    grid=(_GRID,),
    in_specs=[_rowspec(128), _fullspec((128, 128)), _fullspec((128, 16)),
              _fullspec((128, 16))],
    out_specs=_tabs_spec,
    out_shape=_tabs_shape,
)

_mid_in_specs = [_accspec(128), _accspec(16), _fullspec((16, 128)),
                 _fullspec((1, 128)), _fullspec((128, 128)),
                 _fullspec((128, 16)), _fullspec((128, 16))]

_tc_mid_act = pl.pallas_call(
    functools.partial(_tc_mid_body, True),
    grid=(_GRID,), in_specs=_mid_in_specs,
    out_specs=_tabs_spec, out_shape=_tabs_shape,
)

_tc_mid = pl.pallas_call(
    functools.partial(_tc_mid_body, False),
    grid=(_GRID,), in_specs=_mid_in_specs,
    out_specs=_tabs_spec, out_shape=_tabs_shape,
)

_tc_final = pl.pallas_call(
    _tc_final_body,
    grid=(_GRID,),
    in_specs=[_accspec(128), _accspec(16), _fullspec((16, 128)),
              _fullspec((1, 128))],
    out_specs=_rowspec(128),
    out_shape=jax.ShapeDtypeStruct((N_ACC, 128), jnp.float32),
)


def _attn_mats(a_s, a_d):
    if a_s.shape[0] == 1:
        z = jnp.zeros((128, 8), jnp.float32)
        asm = jnp.concatenate([jnp.tile(a_s[0][:, None], (1, 8)), z], axis=1)
        adm = jnp.concatenate([jnp.tile(a_d[0][:, None], (1, 8)), z], axis=1)
    else:
        rows = jnp.arange(128)
        cols = rows // 16
        asm = jnp.zeros((128, 16), jnp.float32).at[rows, cols].set(a_s.reshape(-1))
        adm = jnp.zeros((128, 16), jnp.float32).at[rows, cols].set(a_d.reshape(-1))
    return asm, adm


def kernel(x, edge_index, edge_attr, W_enc, as_enc, ad_enc, b_enc,
           W_h0, as_h0, ad_h0, b_h0, W_h1, as_h1, ad_h1, b_h1,
           W_h2, as_h2, ad_h2, b_h2, W_dec, as_dec, ad_dec, b_dec):
    loop = jnp.arange(N, dtype=jnp.int32)
    pad = jnp.full((E_PAD - E_TOT,), N, dtype=jnp.int32)
    src = jnp.concatenate([edge_index[0], loop, pad])
    dst = jnp.concatenate([edge_index[1], loop, pad])
    ids3 = jnp.stack([src.reshape(NC * NS * CHUNKS, K),
                      dst.reshape(NC * NS * CHUNKS, K)], axis=1)

    e16 = jnp.concatenate(
        [jnp.repeat(jnp.eye(8, dtype=jnp.float32), 16, axis=1),
         jnp.zeros((8, 128), jnp.float32)], axis=0)

    xp = jnp.zeros((N_ACC, 128), jnp.float32).at[:N].set(x)

    layers = [(W_enc, as_enc, ad_enc, b_enc),
              (W_h0, as_h0, ad_h0, b_h0),
              (W_h1, as_h1, ad_h1, b_h1),
              (W_h2, as_h2, ad_h2, b_h2),
              (W_dec, as_dec, ad_dec, b_dec)]
    mats = [_attn_mats(a_s, a_d) for (_, a_s, a_d, _b) in layers]

    h, asv, adv = _tc_enc(xp, W_enc, mats[0][0], mats[0][1])
    for li in range(1, 5):
        num, den = _sc_agg(h, asv, adv, ids3)
        w_next, _, _, b_prev = layers[li][0], None, None, layers[li - 1][3]
        mid = _tc_mid_act if li == 1 else _tc_mid
        h, asv, adv = mid(num, den, e16, b_prev.reshape(1, 128), w_next,
                          mats[li][0], mats[li][1])
    num, den = _sc_agg(h, asv, adv, ids3)
    out = _tc_final(num, den, e16, b_dec.reshape(1, 128))
    return out[:N]


# speedup vs baseline: 1.0570x; 1.0570x over previous
import functools

import jax
import jax.numpy as jnp
from jax import lax
from jax.experimental import pallas as pl
from jax.experimental.pallas import tpu as pltpu
from jax.experimental.pallas import tpu_sc as plsc

N = 10000
IN_DIM = 128
E = 320000

NC, NS = 2, 16
K = 128
E_TOT = E + N
CHUNKS = -(-E_TOT // (NC * NS * K))
C0, C1 = 85, 77
EPT = CHUNKS * K
E_PAD = NC * NS * EPT
ROWS_PT = 632
N_ACC = NS * ROWS_PT
ZSIZES = (128, 128, 128, 128, 120)


def _sc_agg_body(h_hbm, as_hbm, ad_hbm, ids_hbm,
                 num_out, den_out,
                 num_sh, den_sh, idsb, dsc, hrows, msg, asv, adv, wv,
                 sem1, sem2, sem3, sem_sn, sem_sd):
    cid = lax.axis_index("c")
    sid = lax.axis_index("s")
    mychunks = jnp.where(cid == 0, C0, C1)
    blockbase = jnp.where(cid == 0, sid * C0, NS * C0 + sid * C1)

    def _zrow(r, carry):
        zero16 = jnp.zeros((16,), jnp.float32)
        for cb in range(8):
            msg[r, pl.ds(cb * 16, 16)] = zero16
        wv[r, :] = zero16
        return carry
    lax.fori_loop(0, K, _zrow, 0)
    zoff = 0
    for zs in ZSIZES:
        row0 = sid * ROWS_PT + zoff
        pltpu.sync_copy(msg.at[pl.ds(0, zs)], num_sh.at[pl.ds(row0, zs)])
        pltpu.sync_copy(wv.at[pl.ds(0, zs)], den_sh.at[pl.ds(row0, zs)])
        zoff += zs
    plsc.subcore_barrier()

    def _wait_num():
        pltpu.make_async_copy(msg, num_sh.at[dsc], sem_sn).wait()

    def _wait_den():
        pltpu.make_async_copy(wv, den_sh.at[dsc], sem_sd).wait()

    def _chunk(ci, carry):
        pltpu.sync_copy(ids_hbm.at[blockbase + ci], idsb)
        src_v = idsb.at[0]
        dst_v = idsb.at[1]
        cp1 = pltpu.async_copy(h_hbm.at[src_v], hrows, sem1)
        cp2 = pltpu.async_copy(as_hbm.at[src_v], asv, sem2)
        cp3 = pltpu.async_copy(ad_hbm.at[dst_v], adv, sem3)
        @pl.when(ci > 0)
        def _():
            _wait_num()
            _wait_den()
        cp2.wait()
        cp3.wait()
        for t in range(K // 16):
            dsc[pl.ds(t * 16, 16)] = idsb[1, pl.ds(t * 16, 16)]

        def _wrow(k, c):
            e = asv[k, :] + adv[k, :]
            wv[k, :] = jnp.exp(jnp.where(e > 0, e, e * 0.2))
            return c
        lax.fori_loop(0, K, _wrow, 0)
        pltpu.make_async_copy(wv, den_sh.at[dsc], sem_sd).start(add=True)
        cp1.wait()

        def _erow(k, c):
            wrow = wv[k, :]
            for hh in range(8):
                msg[k, pl.ds(hh * 16, 16)] = hrows[k, pl.ds(hh * 16, 16)] * wrow[hh]
            return c
        lax.fori_loop(0, K, _erow, 0)

        pltpu.make_async_copy(msg, num_sh.at[dsc], sem_sn).start(add=True)
        return carry
    lax.fori_loop(0, mychunks, _chunk, 0)
    _wait_num()
    _wait_den()

    plsc.subcore_barrier()
    zoff = 0
    for zs in ZSIZES:
        row0 = sid * ROWS_PT + zoff
        pltpu.sync_copy(num_sh.at[pl.ds(row0, zs)], msg.at[pl.ds(0, zs)])
        pltpu.sync_copy(msg.at[pl.ds(0, zs)], num_out.at[cid, pl.ds(row0, zs)])
        pltpu.sync_copy(den_sh.at[pl.ds(row0, zs)], wv.at[pl.ds(0, zs)])
        pltpu.sync_copy(wv.at[pl.ds(0, zs)], den_out.at[cid, pl.ds(row0, zs)])
        zoff += zs


_sc_agg = functools.partial(
    pl.kernel,
    out_type=[jax.ShapeDtypeStruct((NC, N_ACC, 128), jnp.float32),
              jax.ShapeDtypeStruct((NC, N_ACC, 16), jnp.float32)],
    mesh=plsc.VectorSubcoreMesh(core_axis_name="c", subcore_axis_name="s"),
    compiler_params=pltpu.CompilerParams(use_tc_tiling_on_sc=False),
    scratch_types=[
        pltpu.VMEM_SHARED((N_ACC, 128), jnp.float32),
        pltpu.VMEM_SHARED((N_ACC, 16), jnp.float32),
        pltpu.VMEM((2, K), jnp.int32),
        pltpu.VMEM((K,), jnp.int32),
        pltpu.VMEM((K, 128), jnp.float32),
        pltpu.VMEM((K, 128), jnp.float32),
        pltpu.VMEM((K, 16), jnp.float32),
        pltpu.VMEM((K, 16), jnp.float32),
        pltpu.VMEM((K, 16), jnp.float32),
        pltpu.SemaphoreType.DMA,
        pltpu.SemaphoreType.DMA,
        pltpu.SemaphoreType.DMA,
        pltpu.SemaphoreType.DMA,
        pltpu.SemaphoreType.DMA,
    ],
)(_sc_agg_body)


_GRID = 8
_BLK = N_ACC // _GRID


def _tc_enc_body(x_ref, w_ref, asm_ref, adm_ref, h_ref, as_ref, ad_ref):
    h = jnp.dot(x_ref[...], w_ref[...], preferred_element_type=jnp.float32)
    h_ref[...] = h
    as_ref[...] = jnp.dot(h, asm_ref[...], preferred_element_type=jnp.float32)
    ad_ref[...] = jnp.dot(h, adm_ref[...], preferred_element_type=jnp.float32)


def _merge(num_ref, den_ref, e16_ref, b_ref):
    nsum = num_ref[0] + num_ref[1]
    dsum = den_ref[0] + den_ref[1]
    recip = 1.0 / (dsum + 1e-16)
    rep = jnp.dot(recip, e16_ref[...], preferred_element_type=jnp.float32)
    return nsum * rep + b_ref[...]


def _tc_mid_body(apply_act, num_ref, den_ref, e16_ref, b_ref, w_ref, asm_ref,
                 adm_ref, h_ref, as_ref, ad_ref):
    act = _merge(num_ref, den_ref, e16_ref, b_ref)
    if apply_act:
        act = jnp.where(act > 0, act, act * 0.01)
    h = jnp.dot(act, w_ref[...], preferred_element_type=jnp.float32)
    h_ref[...] = h
    as_ref[...] = jnp.dot(h, asm_ref[...], preferred_element_type=jnp.float32)
    ad_ref[...] = jnp.dot(h, adm_ref[...], preferred_element_type=jnp.float32)


def _tc_final_body(num_ref, den_ref, e16_ref, b_ref, out_ref):
    out_ref[...] = _merge(num_ref, den_ref, e16_ref, b_ref)


def _rowspec(minor):
    return pl.BlockSpec((_BLK, minor), lambda i: (i, 0))


def _accspec(minor):
    return pl.BlockSpec((NC, _BLK, minor), lambda i: (0, i, 0))


def _fullspec(shape):
    return pl.BlockSpec(shape, lambda i: tuple(0 for _ in shape))


_tabs_shape = [jax.ShapeDtypeStruct((N_ACC, 128), jnp.float32),
               jax.ShapeDtypeStruct((N_ACC, 16), jnp.float32),
               jax.ShapeDtypeStruct((N_ACC, 16), jnp.float32)]
_tabs_spec = [_rowspec(128), _rowspec(16), _rowspec(16)]

_tc_enc = pl.pallas_call(
    _tc_enc_body,
    grid=(_GRID,),
    in_specs=[_rowspec(128), _fullspec((128, 128)), _fullspec((128, 16)),
              _fullspec((128, 16))],
    out_specs=_tabs_spec,
    out_shape=_tabs_shape,
)

_mid_in_specs = [_accspec(128), _accspec(16), _fullspec((16, 128)),
                 _fullspec((1, 128)), _fullspec((128, 128)),
                 _fullspec((128, 16)), _fullspec((128, 16))]

_tc_mid_act = pl.pallas_call(
    functools.partial(_tc_mid_body, True),
    grid=(_GRID,), in_specs=_mid_in_specs,
    out_specs=_tabs_spec, out_shape=_tabs_shape,
)

_tc_mid = pl.pallas_call(
    functools.partial(_tc_mid_body, False),
    grid=(_GRID,), in_specs=_mid_in_specs,
    out_specs=_tabs_spec, out_shape=_tabs_shape,
)

_tc_final = pl.pallas_call(
    _tc_final_body,
    grid=(_GRID,),
    in_specs=[_accspec(128), _accspec(16), _fullspec((16, 128)),
              _fullspec((1, 128))],
    out_specs=_rowspec(128),
    out_shape=jax.ShapeDtypeStruct((N_ACC, 128), jnp.float32),
)


def _attn_mats(a_s, a_d):
    if a_s.shape[0] == 1:
        z = jnp.zeros((128, 8), jnp.float32)
        asm = jnp.concatenate([jnp.tile(a_s[0][:, None], (1, 8)), z], axis=1)
        adm = jnp.concatenate([jnp.tile(a_d[0][:, None], (1, 8)), z], axis=1)
    else:
        rows = jnp.arange(128)
        cols = rows // 16
        asm = jnp.zeros((128, 16), jnp.float32).at[rows, cols].set(a_s.reshape(-1))
        adm = jnp.zeros((128, 16), jnp.float32).at[rows, cols].set(a_d.reshape(-1))
    return asm, adm


def kernel(x, edge_index, edge_attr, W_enc, as_enc, ad_enc, b_enc,
           W_h0, as_h0, ad_h0, b_h0, W_h1, as_h1, ad_h1, b_h1,
           W_h2, as_h2, ad_h2, b_h2, W_dec, as_dec, ad_dec, b_dec):
    loop = jnp.arange(N, dtype=jnp.int32)
    pad = jnp.full((E_PAD - E_TOT,), N, dtype=jnp.int32)
    src = jnp.concatenate([edge_index[0], loop, pad])
    dst = jnp.concatenate([edge_index[1], loop, pad])
    ids3 = jnp.stack([src.reshape(NC * NS * CHUNKS, K),
                      dst.reshape(NC * NS * CHUNKS, K)], axis=1)

    e16 = jnp.concatenate(
        [jnp.repeat(jnp.eye(8, dtype=jnp.float32), 16, axis=1),
         jnp.zeros((8, 128), jnp.float32)], axis=0)

    xp = jnp.zeros((N_ACC, 128), jnp.float32).at[:N].set(x)

    layers = [(W_enc, as_enc, ad_enc, b_enc),
              (W_h0, as_h0, ad_h0, b_h0),
              (W_h1, as_h1, ad_h1, b_h1),
              (W_h2, as_h2, ad_h2, b_h2),
              (W_dec, as_dec, ad_dec, b_dec)]
    mats = [_attn_mats(a_s, a_d) for (_, a_s, a_d, _b) in layers]

    h, asv, adv = _tc_enc(xp, W_enc, mats[0][0], mats[0][1])
    for li in range(1, 5):
        num, den = _sc_agg(h, asv, adv, ids3)
        w_next, _, _, b_prev = layers[li][0], None, None, layers[li - 1][3]
        mid = _tc_mid_act if li == 1 else _tc_mid
        h, asv, adv = mid(num, den, e16, b_prev.reshape(1, 128), w_next,
                          mats[li][0], mats[li][1])
    num, den = _sc_agg(h, asv, adv, ids3)
    out = _tc_final(num, den, e16, b_dec.reshape(1, 128))
    return out[:N]


# speedup vs baseline: 1.0799x; 1.0217x over previous
import functools

import jax
import jax.numpy as jnp
from jax import lax
from jax.experimental import pallas as pl
from jax.experimental.pallas import tpu as pltpu
from jax.experimental.pallas import tpu_sc as plsc

N = 10000
IN_DIM = 128
E = 320000

NC, NS = 2, 16
K = 128
E_TOT = E + N
CHUNKS = -(-E_TOT // (NC * NS * K))
C0, C1 = 88, 74
EPT = CHUNKS * K
E_PAD = NC * NS * EPT
ROWS_PT = 632
N_ACC = NS * ROWS_PT
ZSIZES = (128, 128, 128, 128, 120)


def _sc_agg_body(h_hbm, as_hbm, ad_hbm, ids_hbm,
                 num_out, den_out,
                 num_sh, den_sh, idsb, dsc, hrows, msg, asv, adv, wv,
                 sem1, sem2, sem3, sem_sn, sem_sd):
    cid = lax.axis_index("c")
    sid = lax.axis_index("s")
    mychunks = jnp.where(cid == 0, C0, C1)
    blockbase = jnp.where(cid == 0, sid * C0, NS * C0 + sid * C1)

    def _zrow(r, carry):
        zero16 = jnp.zeros((16,), jnp.float32)
        for cb in range(8):
            msg[r, pl.ds(cb * 16, 16)] = zero16
        wv[r, :] = zero16
        return carry
    lax.fori_loop(0, K, _zrow, 0)
    zoff = 0
    for zs in ZSIZES:
        row0 = sid * ROWS_PT + zoff
        pltpu.sync_copy(msg.at[pl.ds(0, zs)], num_sh.at[pl.ds(row0, zs)])
        pltpu.sync_copy(wv.at[pl.ds(0, zs)], den_sh.at[pl.ds(row0, zs)])
        zoff += zs
    plsc.subcore_barrier()

    def _wait_num():
        pltpu.make_async_copy(msg, num_sh.at[dsc], sem_sn).wait()

    def _wait_den():
        pltpu.make_async_copy(wv, den_sh.at[dsc], sem_sd).wait()

    def _chunk(ci, carry):
        pltpu.sync_copy(ids_hbm.at[blockbase + ci], idsb)
        src_v = idsb.at[0]
        dst_v = idsb.at[1]
        cp1 = pltpu.async_copy(h_hbm.at[src_v], hrows, sem1)
        cp2 = pltpu.async_copy(as_hbm.at[src_v], asv, sem2)
        cp3 = pltpu.async_copy(ad_hbm.at[dst_v], adv, sem3)
        @pl.when(ci > 0)
        def _():
            _wait_num()
            _wait_den()
        cp2.wait()
        cp3.wait()
        for t in range(K // 16):
            dsc[pl.ds(t * 16, 16)] = idsb[1, pl.ds(t * 16, 16)]

        def _wrow(k, c):
            e = asv[k, :] + adv[k, :]
            wv[k, :] = jnp.exp(jnp.where(e > 0, e, e * 0.2))
            return c
        lax.fori_loop(0, K, _wrow, 0)
        pltpu.make_async_copy(wv, den_sh.at[dsc], sem_sd).start(add=True)
        cp1.wait()

        def _erow(k, c):
            wrow = wv[k, :]
            for hh in range(8):
                msg[k, pl.ds(hh * 16, 16)] = hrows[k, pl.ds(hh * 16, 16)] * wrow[hh]
            return c
        lax.fori_loop(0, K, _erow, 0)

        pltpu.make_async_copy(msg, num_sh.at[dsc], sem_sn).start(add=True)
        return carry
    lax.fori_loop(0, mychunks, _chunk, 0)
    _wait_num()
    _wait_den()

    plsc.subcore_barrier()
    zoff = 0
    for zs in ZSIZES:
        row0 = sid * ROWS_PT + zoff
        pltpu.sync_copy(num_sh.at[pl.ds(row0, zs)], msg.at[pl.ds(0, zs)])
        pltpu.sync_copy(msg.at[pl.ds(0, zs)], num_out.at[cid, pl.ds(row0, zs)])
        pltpu.sync_copy(den_sh.at[pl.ds(row0, zs)], wv.at[pl.ds(0, zs)])
        pltpu.sync_copy(wv.at[pl.ds(0, zs)], den_out.at[cid, pl.ds(row0, zs)])
        zoff += zs


_sc_agg = functools.partial(
    pl.kernel,
    out_type=[jax.ShapeDtypeStruct((NC, N_ACC, 128), jnp.float32),
              jax.ShapeDtypeStruct((NC, N_ACC, 16), jnp.float32)],
    mesh=plsc.VectorSubcoreMesh(core_axis_name="c", subcore_axis_name="s"),
    compiler_params=pltpu.CompilerParams(use_tc_tiling_on_sc=False),
    scratch_types=[
        pltpu.VMEM_SHARED((N_ACC, 128), jnp.float32),
        pltpu.VMEM_SHARED((N_ACC, 16), jnp.float32),
        pltpu.VMEM((2, K), jnp.int32),
        pltpu.VMEM((K,), jnp.int32),
        pltpu.VMEM((K, 128), jnp.float32),
        pltpu.VMEM((K, 128), jnp.float32),
        pltpu.VMEM((K, 16), jnp.float32),
        pltpu.VMEM((K, 16), jnp.float32),
        pltpu.VMEM((K, 16), jnp.float32),
        pltpu.SemaphoreType.DMA,
        pltpu.SemaphoreType.DMA,
        pltpu.SemaphoreType.DMA,
        pltpu.SemaphoreType.DMA,
        pltpu.SemaphoreType.DMA,
    ],
)(_sc_agg_body)


_GRID = 8
_BLK = N_ACC // _GRID


def _tc_enc_body(x_ref, w_ref, asm_ref, adm_ref, h_ref, as_ref, ad_ref):
    h = jnp.dot(x_ref[...], w_ref[...], preferred_element_type=jnp.float32)
    h_ref[...] = h
    as_ref[...] = jnp.dot(h, asm_ref[...], preferred_element_type=jnp.float32)
    ad_ref[...] = jnp.dot(h, adm_ref[...], preferred_element_type=jnp.float32)


def _merge(num_ref, den_ref, e16_ref, b_ref):
    nsum = num_ref[0] + num_ref[1]
    dsum = den_ref[0] + den_ref[1]
    recip = 1.0 / (dsum + 1e-16)
    rep = jnp.dot(recip, e16_ref[...], preferred_element_type=jnp.float32)
    return nsum * rep + b_ref[...]


def _tc_mid_body(apply_act, num_ref, den_ref, e16_ref, b_ref, w_ref, asm_ref,
                 adm_ref, h_ref, as_ref, ad_ref):
    act = _merge(num_ref, den_ref, e16_ref, b_ref)
    if apply_act:
        act = jnp.where(act > 0, act, act * 0.01)
    h = jnp.dot(act, w_ref[...], preferred_element_type=jnp.float32)
    h_ref[...] = h
    as_ref[...] = jnp.dot(h, asm_ref[...], preferred_element_type=jnp.float32)
    ad_ref[...] = jnp.dot(h, adm_ref[...], preferred_element_type=jnp.float32)


def _tc_final_body(num_ref, den_ref, e16_ref, b_ref, out_ref):
    out_ref[...] = _merge(num_ref, den_ref, e16_ref, b_ref)


def _rowspec(minor):
    return pl.BlockSpec((_BLK, minor), lambda i: (i, 0))


def _accspec(minor):
    return pl.BlockSpec((NC, _BLK, minor), lambda i: (0, i, 0))


def _fullspec(shape):
    return pl.BlockSpec(shape, lambda i: tuple(0 for _ in shape))


_tabs_shape = [jax.ShapeDtypeStruct((N_ACC, 128), jnp.float32),
               jax.ShapeDtypeStruct((N_ACC, 16), jnp.float32),
               jax.ShapeDtypeStruct((N_ACC, 16), jnp.float32)]
_tabs_spec = [_rowspec(128), _rowspec(16), _rowspec(16)]

_tc_enc = pl.pallas_call(
    _tc_enc_body,
    grid=(_GRID,),
    in_specs=[_rowspec(128), _fullspec((128, 128)), _fullspec((128, 16)),
              _fullspec((128, 16))],
    out_specs=_tabs_spec,
    out_shape=_tabs_shape,
)

_mid_in_specs = [_accspec(128), _accspec(16), _fullspec((16, 128)),
                 _fullspec((1, 128)), _fullspec((128, 128)),
                 _fullspec((128, 16)), _fullspec((128, 16))]

_tc_mid_act = pl.pallas_call(
    functools.partial(_tc_mid_body, True),
    grid=(_GRID,), in_specs=_mid_in_specs,
    out_specs=_tabs_spec, out_shape=_tabs_shape,
)

_tc_mid = pl.pallas_call(
    functools.partial(_tc_mid_body, False),
    grid=(_GRID,), in_specs=_mid_in_specs,
    out_specs=_tabs_spec, out_shape=_tabs_shape,
)

_tc_final = pl.pallas_call(
    _tc_final_body,
    grid=(_GRID,),
    in_specs=[_accspec(128), _accspec(16), _fullspec((16, 128)),
              _fullspec((1, 128))],
    out_specs=_rowspec(128),
    out_shape=jax.ShapeDtypeStruct((N_ACC, 128), jnp.float32),
)


def _attn_mats(a_s, a_d):
    if a_s.shape[0] == 1:
        z = jnp.zeros((128, 8), jnp.float32)
        asm = jnp.concatenate([jnp.tile(a_s[0][:, None], (1, 8)), z], axis=1)
        adm = jnp.concatenate([jnp.tile(a_d[0][:, None], (1, 8)), z], axis=1)
    else:
        rows = jnp.arange(128)
        cols = rows // 16
        asm = jnp.zeros((128, 16), jnp.float32).at[rows, cols].set(a_s.reshape(-1))
        adm = jnp.zeros((128, 16), jnp.float32).at[rows, cols].set(a_d.reshape(-1))
    return asm, adm


def kernel(x, edge_index, edge_attr, W_enc, as_enc, ad_enc, b_enc,
           W_h0, as_h0, ad_h0, b_h0, W_h1, as_h1, ad_h1, b_h1,
           W_h2, as_h2, ad_h2, b_h2, W_dec, as_dec, ad_dec, b_dec):
    loop = jnp.arange(N, dtype=jnp.int32)
    pad = jnp.full((E_PAD - E_TOT,), N, dtype=jnp.int32)
    src = jnp.concatenate([edge_index[0], loop, pad])
    dst = jnp.concatenate([edge_index[1], loop, pad])
    ids3 = jnp.stack([src.reshape(NC * NS * CHUNKS, K),
                      dst.reshape(NC * NS * CHUNKS, K)], axis=1)

    e16 = jnp.concatenate(
        [jnp.repeat(jnp.eye(8, dtype=jnp.float32), 16, axis=1),
         jnp.zeros((8, 128), jnp.float32)], axis=0)

    xp = jnp.zeros((N_ACC, 128), jnp.float32).at[:N].set(x)

    layers = [(W_enc, as_enc, ad_enc, b_enc),
              (W_h0, as_h0, ad_h0, b_h0),
              (W_h1, as_h1, ad_h1, b_h1),
              (W_h2, as_h2, ad_h2, b_h2),
              (W_dec, as_dec, ad_dec, b_dec)]
    mats = [_attn_mats(a_s, a_d) for (_, a_s, a_d, _b) in layers]

    h, asv, adv = _tc_enc(xp, W_enc, mats[0][0], mats[0][1])
    for li in range(1, 5):
        num, den = _sc_agg(h, asv, adv, ids3)
        w_next, _, _, b_prev = layers[li][0], None, None, layers[li - 1][3]
        mid = _tc_mid_act if li == 1 else _tc_mid
        h, asv, adv = mid(num, den, e16, b_prev.reshape(1, 128), w_next,
                          mats[li][0], mats[li][1])
    num, den = _sc_agg(h, asv, adv, ids3)
    out = _tc_final(num, den, e16, b_dec.reshape(1, 128))
    return out[:N]
